# baseline (speedup 1.0000x reference)
import functools

import jax
import jax.numpy as jnp
from jax import lax
from jax.experimental import pallas as pl
from jax.experimental.pallas import tpu as pltpu
from jax.experimental.pallas import tpu_sc as plsc

_COMMITMENT_COST = 0.25
_BLOCK = 4096
_NC, _NS, _L = 2, 16, 16
_NW = _NC * _NS


def _vq_argmin_block(x_ref, wsq_ref, wt_ref, idx_ref, loss_ref):
    x = x_ref[...]
    wt = wt_ref[...]
    x_sq = jnp.sum(x * x, axis=1, keepdims=True)
    dot = jnp.dot(x, wt, preferred_element_type=jnp.float32)
    d = (x_sq + wsq_ref[...]) - 2.0 * dot
    m = jnp.min(d, axis=1, keepdims=True)
    cols = lax.broadcasted_iota(jnp.int32, d.shape, 1)
    idx = jnp.min(jnp.where(d == m, cols, d.shape[1]), axis=1)
    idx_ref[...] = idx.reshape(idx_ref.shape)

    @pl.when(pl.program_id(0) == 0)
    def _init():
        loss_ref[0, 0] = 0.0

    loss_ref[0, 0] += jnp.sum(m)


def _tc_argmin(flat, w_sq, w_t):
    n, d = flat.shape
    k = w_t.shape[1]
    grid = n // _BLOCK
    rows = _BLOCK // 128
    return pl.pallas_call(
        _vq_argmin_block,
        grid=(grid,),
        in_specs=[
            pl.BlockSpec((_BLOCK, d), lambda i: (i, 0)),
            pl.BlockSpec((1, k), lambda i: (0, 0)),
            pl.BlockSpec((d, k), lambda i: (0, 0)),
        ],
        out_specs=[
            pl.BlockSpec((rows, 128), lambda i: (i, 0)),
            pl.BlockSpec((1, 1), lambda i: (0, 0), memory_space=pltpu.SMEM),
        ],
        out_shape=[
            jax.ShapeDtypeStruct((n // 128, 128), jnp.int32),
            jax.ShapeDtypeStruct((1, 1), jnp.float32),
        ],
    )(flat, w_sq, w_t)


def _sc_gather(weight, idx):
    n = idx.shape[0]
    d = weight.shape[1]
    b_per_w = n // _NW
    mesh = plsc.VectorSubcoreMesh(core_axis_name="c", subcore_axis_name="s")

    @functools.partial(
        pl.kernel, mesh=mesh,
        compiler_params=pltpu.CompilerParams(
            needs_layout_passes=False, use_tc_tiling_on_sc=False),
        out_type=jax.ShapeDtypeStruct((n, d), jnp.float32),
        scratch_types=[
            pltpu.VMEM((b_per_w,), jnp.int32),
            pltpu.VMEM((b_per_w, d), jnp.float32),
            pltpu.SemaphoreType.DMA,
        ],
    )
    def k(table_hbm, idx_hbm, out_hbm, idx_v, rows_v, sem):
        wid = lax.axis_index("s") * _NC + lax.axis_index("c")
        base = wid * b_per_w
        pltpu.sync_copy(idx_hbm.at[pl.ds(base, b_per_w)], idx_v)
        pltpu.async_copy(table_hbm.at[idx_v], rows_v, sem).wait()
        pltpu.sync_copy(rows_v, out_hbm.at[pl.ds(base, b_per_w)])

    return k(weight, idx)


@jax.jit
def _vq(inputs, weight):
    d = weight.shape[1]
    flat = inputs.reshape(-1, d)
    n = flat.shape[0]
    w_sq = jnp.sum(weight ** 2, axis=1).reshape(1, -1)
    w_t = weight.T
    h = n // 2
    halves = []
    loss_sum = jnp.float32(0)
    for s in range(2):
        part = lax.slice_in_dim(flat, s * h, (s + 1) * h, axis=0)
        idx, lsum = _tc_argmin(part, w_sq, w_t)
        halves.append(_sc_gather(weight, idx.reshape(-1)))
        loss_sum = loss_sum + lsum[0, 0]
    q = jnp.concatenate(halves, axis=0)
    mean_sq = loss_sum / jnp.float32(flat.size)
    return (q.reshape(inputs.shape), mean_sq, _COMMITMENT_COST * mean_sq)


def kernel(inputs, weight):
    return _vq(inputs, weight)


# speedup vs baseline: 1.3858x; 1.3858x over previous
import functools

import jax
import jax.numpy as jnp
from jax import lax
from jax.experimental import pallas as pl
from jax.experimental.pallas import tpu as pltpu
from jax.experimental.pallas import tpu_sc as plsc

_COMMITMENT_COST = 0.25
_NC, _NS, _L = 2, 16, 16
_NW = _NC * _NS


def _vq_argmin_block(xt_ref, w_ref, idx_ref, loss_ref):
    xt = xt_ref[0]
    w = w_ref[...]
    x_sq = jnp.sum(xt * xt, axis=0, keepdims=True)
    w_sq = jnp.sum(w * w, axis=1, keepdims=True)
    dot = jnp.dot(w, xt, preferred_element_type=jnp.float32)
    d = (x_sq + w_sq) - 2.0 * dot
    m = jnp.min(d, axis=0, keepdims=True)
    rows = lax.broadcasted_iota(jnp.int32, d.shape, 0)
    idx = jnp.min(jnp.where(d == m, rows, d.shape[0]),
                  axis=0, keepdims=True)
    idx_ref[...] = idx.reshape(idx_ref.shape)

    @pl.when(pl.program_id(0) == 0)
    def _init():
        loss_ref[0, 0] = 0.0

    loss_ref[0, 0] += jnp.sum(m)


def _tc_argmin(xt, w):
    b, d, p = xt.shape
    k = w.shape[0]
    return pl.pallas_call(
        _vq_argmin_block,
        grid=(b,),
        in_specs=[
            pl.BlockSpec((1, d, p), lambda i: (i, 0, 0)),
            pl.BlockSpec((k, d), lambda i: (0, 0)),
        ],
        out_specs=[
            pl.BlockSpec((1, 1, p), lambda i: (i, 0, 0)),
            pl.BlockSpec((1, 1), lambda i: (0, 0), memory_space=pltpu.SMEM),
        ],
        out_shape=[
            jax.ShapeDtypeStruct((b, 1, p), jnp.int32),
            jax.ShapeDtypeStruct((1, 1), jnp.float32),
        ],
    )(xt, w)


def _sc_gather(weight, idx):
    n = idx.shape[0]
    d = weight.shape[1]
    b_per_w = n // _NW
    mesh = plsc.VectorSubcoreMesh(core_axis_name="c", subcore_axis_name="s")

    @functools.partial(
        pl.kernel, mesh=mesh,
        compiler_params=pltpu.CompilerParams(
            needs_layout_passes=False, use_tc_tiling_on_sc=False),
        out_type=jax.ShapeDtypeStruct((n, d), jnp.float32),
        scratch_types=[
            pltpu.VMEM((b_per_w,), jnp.int32),
            pltpu.VMEM((b_per_w, d), jnp.float32),
            pltpu.SemaphoreType.DMA,
        ],
    )
    def k(table_hbm, idx_hbm, out_hbm, idx_v, rows_v, sem):
        wid = lax.axis_index("s") * _NC + lax.axis_index("c")
        base = wid * b_per_w
        pltpu.sync_copy(idx_hbm.at[pl.ds(base, b_per_w)], idx_v)
        pltpu.async_copy(table_hbm.at[idx_v], rows_v, sem).wait()
        pltpu.sync_copy(rows_v, out_hbm.at[pl.ds(base, b_per_w)])

    return k(weight, idx)


@jax.jit
def _vq(inputs, weight):
    d = weight.shape[1]
    xt = inputs.transpose(0, 2, 1)
    idx, loss_sum = _tc_argmin(xt, weight)
    q = _sc_gather(weight, idx.reshape(-1))
    mean_sq = loss_sum[0, 0] / jnp.float32(inputs.size)
    return (q.reshape(inputs.shape), mean_sq, _COMMITMENT_COST * mean_sq)


def kernel(inputs, weight):
    return _vq(inputs, weight)


# speedup vs baseline: 1.4798x; 1.0678x over previous
import functools

import jax
import jax.numpy as jnp
from jax import lax
from jax.experimental import pallas as pl
from jax.experimental.pallas import tpu as pltpu
from jax.experimental.pallas import tpu_sc as plsc

_COMMITMENT_COST = 0.25
_NC, _NS, _L = 2, 16, 16
_NW = _NC * _NS


def _vq_argmin_block(xt_ref, w_ref, idx_ref, loss_ref):
    w = w_ref[...]
    w_sq = jnp.sum(w * w, axis=1, keepdims=True)

    @pl.when(pl.program_id(0) == 0)
    def _init():
        loss_ref[0, 0] = 0.0

    part = jnp.float32(0)
    for b in range(xt_ref.shape[0]):
        xt = xt_ref[b]
        x_sq = jnp.sum(xt * xt, axis=0, keepdims=True)
        dot = jnp.dot(w, xt, preferred_element_type=jnp.float32)
        d = (x_sq + w_sq) - 2.0 * dot
        m = jnp.min(d, axis=0, keepdims=True)
        rows = lax.broadcasted_iota(jnp.int32, d.shape, 0)
        idx = jnp.min(jnp.where(d == m, rows, d.shape[0]),
                      axis=0, keepdims=True)
        idx_ref[b] = idx
        part = part + jnp.sum(m)

    loss_ref[0, 0] += part


_BB = 2


def _tc_argmin(xt, w):
    b, d, p = xt.shape
    k = w.shape[0]
    return pl.pallas_call(
        _vq_argmin_block,
        grid=(b // _BB,),
        in_specs=[
            pl.BlockSpec((_BB, d, p), lambda i: (i, 0, 0)),
            pl.BlockSpec((k, d), lambda i: (0, 0)),
        ],
        out_specs=[
            pl.BlockSpec((_BB, 1, p), lambda i: (i, 0, 0)),
            pl.BlockSpec((1, 1), lambda i: (0, 0), memory_space=pltpu.SMEM),
        ],
        out_shape=[
            jax.ShapeDtypeStruct((b, 1, p), jnp.int32),
            jax.ShapeDtypeStruct((1, 1), jnp.float32),
        ],
    )(xt, w)


def _sc_gather(weight, idx):
    n = idx.shape[0]
    d = weight.shape[1]
    b_per_w = n // _NW
    mesh = plsc.VectorSubcoreMesh(core_axis_name="c", subcore_axis_name="s")

    @functools.partial(
        pl.kernel, mesh=mesh,
        compiler_params=pltpu.CompilerParams(
            needs_layout_passes=False, use_tc_tiling_on_sc=False),
        out_type=jax.ShapeDtypeStruct((n, d), jnp.float32),
        scratch_types=[
            pltpu.VMEM((b_per_w,), jnp.int32),
            pltpu.VMEM((b_per_w, d), jnp.float32),
            pltpu.SemaphoreType.DMA,
        ],
    )
    def k(table_hbm, idx_hbm, out_hbm, idx_v, rows_v, sem):
        wid = lax.axis_index("s") * _NC + lax.axis_index("c")
        base = wid * b_per_w
        pltpu.sync_copy(idx_hbm.at[pl.ds(base, b_per_w)], idx_v)
        pltpu.async_copy(table_hbm.at[idx_v], rows_v, sem).wait()
        pltpu.sync_copy(rows_v, out_hbm.at[pl.ds(base, b_per_w)])

    return k(weight, idx)


@jax.jit
def _vq(inputs, weight):
    d = weight.shape[1]
    xt = inputs.transpose(0, 2, 1)
    idx, loss_sum = _tc_argmin(xt, weight)
    q = _sc_gather(weight, idx.reshape(-1))
    mean_sq = loss_sum[0, 0] / jnp.float32(inputs.size)
    return (q.reshape(inputs.shape), mean_sq, _COMMITMENT_COST * mean_sq)


def kernel(inputs, weight):
    return _vq(inputs, weight)


# speedup vs baseline: 1.5276x; 1.0323x over previous
import functools

import jax
import jax.numpy as jnp
from jax import lax
from jax.experimental import pallas as pl
from jax.experimental.pallas import tpu as pltpu
from jax.experimental.pallas import tpu_sc as plsc

_COMMITMENT_COST = 0.25
_NC, _NS, _L = 2, 16, 16
_NW = _NC * _NS


def _vq_argmin_block(xt_ref, w_ref, idx_ref, loss_ref):
    w = w_ref[...]
    w_sq = jnp.sum(w * w, axis=1, keepdims=True)

    @pl.when(pl.program_id(0) == 0)
    def _init():
        loss_ref[0, 0] = 0.0

    part = jnp.float32(0)
    for b in range(xt_ref.shape[0]):
        xt = xt_ref[b]
        x_sq = jnp.sum(xt * xt, axis=0, keepdims=True)
        dot = jnp.dot(w, xt, preferred_element_type=jnp.float32)
        d = (x_sq + w_sq) - 2.0 * dot
        m = jnp.min(d, axis=0, keepdims=True)
        rows = lax.broadcasted_iota(jnp.int32, d.shape, 0)
        idx = jnp.min(jnp.where(d == m, rows, d.shape[0]),
                      axis=0, keepdims=True)
        idx_ref[b] = idx
        part = part + jnp.sum(m)

    loss_ref[0, 0] += part


_BB = 4


def _tc_argmin(xt, w):
    b, d, p = xt.shape
    k = w.shape[0]
    return pl.pallas_call(
        _vq_argmin_block,
        grid=(b // _BB,),
        in_specs=[
            pl.BlockSpec((_BB, d, p), lambda i: (i, 0, 0)),
            pl.BlockSpec((k, d), lambda i: (0, 0)),
        ],
        out_specs=[
            pl.BlockSpec((_BB, 1, p), lambda i: (i, 0, 0)),
            pl.BlockSpec((1, 1), lambda i: (0, 0), memory_space=pltpu.SMEM),
        ],
        out_shape=[
            jax.ShapeDtypeStruct((b, 1, p), jnp.int32),
            jax.ShapeDtypeStruct((1, 1), jnp.float32),
        ],
    )(xt, w)


def _sc_gather(weight, idx):
    n = idx.shape[0]
    d = weight.shape[1]
    b_per_w = n // _NW
    mesh = plsc.VectorSubcoreMesh(core_axis_name="c", subcore_axis_name="s")

    @functools.partial(
        pl.kernel, mesh=mesh,
        compiler_params=pltpu.CompilerParams(
            needs_layout_passes=False, use_tc_tiling_on_sc=False),
        out_type=jax.ShapeDtypeStruct((n, d), jnp.float32),
        scratch_types=[
            pltpu.VMEM((b_per_w,), jnp.int32),
            pltpu.VMEM((b_per_w, d), jnp.float32),
            pltpu.SemaphoreType.DMA,
        ],
    )
    def k(table_hbm, idx_hbm, out_hbm, idx_v, rows_v, sem):
        wid = lax.axis_index("s") * _NC + lax.axis_index("c")
        base = wid * b_per_w
        pltpu.sync_copy(idx_hbm.at[pl.ds(base, b_per_w)], idx_v)
        pltpu.async_copy(table_hbm.at[idx_v], rows_v, sem).wait()
        pltpu.sync_copy(rows_v, out_hbm.at[pl.ds(base, b_per_w)])

    return k(weight, idx)


@jax.jit
def _vq(inputs, weight):
    d = weight.shape[1]
    xt = inputs.transpose(0, 2, 1)
    idx, loss_sum = _tc_argmin(xt, weight)
    q = _sc_gather(weight, idx.reshape(-1))
    mean_sq = loss_sum[0, 0] / jnp.float32(inputs.size)
    return (q.reshape(inputs.shape), mean_sq, _COMMITMENT_COST * mean_sq)


def kernel(inputs, weight):
    return _vq(inputs, weight)


# speedup vs baseline: 1.5366x; 1.0059x over previous
import functools

import jax
import jax.numpy as jnp
from jax import lax
from jax.experimental import pallas as pl
from jax.experimental.pallas import tpu as pltpu
from jax.experimental.pallas import tpu_sc as plsc

_COMMITMENT_COST = 0.25
_NC, _NS, _L = 2, 16, 16
_NW = _NC * _NS


def _vq_argmin_block(xt_ref, w_ref, idx_ref, loss_ref):
    w = w_ref[...]
    w_sq = jnp.sum(w * w, axis=1, keepdims=True)

    @pl.when(pl.program_id(0) == 0)
    def _init():
        loss_ref[0, 0] = 0.0

    part = jnp.float32(0)
    for b in range(xt_ref.shape[0]):
        xt = xt_ref[b]
        x_sq = jnp.sum(xt * xt, axis=0, keepdims=True)
        dot = jnp.dot(w, xt, preferred_element_type=jnp.float32)
        d = (x_sq + w_sq) - 2.0 * dot
        m = jnp.min(d, axis=0, keepdims=True)
        rows = lax.broadcasted_iota(jnp.int32, d.shape, 0)
        idx = jnp.min(jnp.where(d == m, rows, d.shape[0]),
                      axis=0, keepdims=True)
        idx_ref[b] = idx
        part = part + jnp.sum(m)

    loss_ref[0, 0] += part


_BB = 8


def _tc_argmin(xt, w):
    b, d, p = xt.shape
    k = w.shape[0]
    return pl.pallas_call(
        _vq_argmin_block,
        grid=(b // _BB,),
        in_specs=[
            pl.BlockSpec((_BB, d, p), lambda i: (i, 0, 0)),
            pl.BlockSpec((k, d), lambda i: (0, 0)),
        ],
        out_specs=[
            pl.BlockSpec((_BB, 1, p), lambda i: (i, 0, 0)),
            pl.BlockSpec((1, 1), lambda i: (0, 0), memory_space=pltpu.SMEM),
        ],
        out_shape=[
            jax.ShapeDtypeStruct((b, 1, p), jnp.int32),
            jax.ShapeDtypeStruct((1, 1), jnp.float32),
        ],
    )(xt, w)


def _sc_gather(weight, idx):
    n = idx.shape[0]
    d = weight.shape[1]
    b_per_w = n // _NW
    mesh = plsc.VectorSubcoreMesh(core_axis_name="c", subcore_axis_name="s")

    half = b_per_w // 2

    @functools.partial(
        pl.kernel, mesh=mesh,
        compiler_params=pltpu.CompilerParams(
            needs_layout_passes=False, use_tc_tiling_on_sc=False),
        out_type=jax.ShapeDtypeStruct((n, d), jnp.float32),
        scratch_types=[
            pltpu.VMEM((b_per_w,), jnp.int32),
            pltpu.VMEM((half, d), jnp.float32),
            pltpu.VMEM((half, d), jnp.float32),
            pltpu.SemaphoreType.DMA,
            pltpu.SemaphoreType.DMA,
            pltpu.SemaphoreType.DMA,
            pltpu.SemaphoreType.DMA,
        ],
    )
    def k(table_hbm, idx_hbm, out_hbm, idx_v, rows_a, rows_b,
          sem_a, sem_b, sem_wa, sem_wb):
        wid = lax.axis_index("s") * _NC + lax.axis_index("c")
        base = wid * b_per_w
        pltpu.sync_copy(idx_hbm.at[pl.ds(base, b_per_w)], idx_v)
        ga = pltpu.async_copy(
            table_hbm.at[idx_v.at[pl.ds(0, half)]], rows_a, sem_a)
        gb = pltpu.async_copy(
            table_hbm.at[idx_v.at[pl.ds(half, half)]], rows_b, sem_b)
        ga.wait()
        wa = pltpu.async_copy(
            rows_a, out_hbm.at[pl.ds(base, half)], sem_wa)
        gb.wait()
        wb = pltpu.async_copy(
            rows_b, out_hbm.at[pl.ds(base + half, half)], sem_wb)
        wa.wait()
        wb.wait()

    return k(weight, idx)


@jax.jit
def _vq(inputs, weight):
    d = weight.shape[1]
    xt = inputs.transpose(0, 2, 1)
    idx, loss_sum = _tc_argmin(xt, weight)
    q = _sc_gather(weight, idx.reshape(-1))
    mean_sq = loss_sum[0, 0] / jnp.float32(inputs.size)
    return (q.reshape(inputs.shape), mean_sq, _COMMITMENT_COST * mean_sq)


def kernel(inputs, weight):
    return _vq(inputs, weight)


# speedup vs baseline: 1.5457x; 1.0060x over previous
import functools

import jax
import jax.numpy as jnp
from jax import lax
from jax.experimental import pallas as pl
from jax.experimental.pallas import tpu as pltpu
from jax.experimental.pallas import tpu_sc as plsc

_COMMITMENT_COST = 0.25
_NC, _NS, _L = 2, 16, 16
_NW = _NC * _NS


def _vq_argmin_block(xt_ref, w_ref, idx_ref, loss_ref):
    w = w_ref[...]
    w_sq = jnp.sum(w * w, axis=1, keepdims=True)

    @pl.when(pl.program_id(0) == 0)
    def _init():
        loss_ref[0, 0] = 0.0

    part = jnp.float32(0)
    for b in range(xt_ref.shape[0]):
        xt = xt_ref[b]
        x_sq = jnp.sum(xt * xt, axis=0, keepdims=True)
        dot = jnp.dot(w, xt, preferred_element_type=jnp.float32)
        d = (x_sq + w_sq) - 2.0 * dot
        m = jnp.min(d, axis=0, keepdims=True)
        rows = lax.broadcasted_iota(jnp.int32, d.shape, 0)
        idx = jnp.min(jnp.where(d == m, rows, d.shape[0]),
                      axis=0, keepdims=True)
        idx_ref[b] = idx
        part = part + jnp.sum(m)

    loss_ref[0, 0] += part


_BB = 16


def _tc_argmin(xt, w):
    b, d, p = xt.shape
    k = w.shape[0]
    return pl.pallas_call(
        _vq_argmin_block,
        grid=(b // _BB,),
        in_specs=[
            pl.BlockSpec((_BB, d, p), lambda i: (i, 0, 0)),
            pl.BlockSpec((k, d), lambda i: (0, 0)),
        ],
        out_specs=[
            pl.BlockSpec((_BB, 1, p), lambda i: (i, 0, 0)),
            pl.BlockSpec((1, 1), lambda i: (0, 0), memory_space=pltpu.SMEM),
        ],
        out_shape=[
            jax.ShapeDtypeStruct((b, 1, p), jnp.int32),
            jax.ShapeDtypeStruct((1, 1), jnp.float32),
        ],
    )(xt, w)


def _sc_gather(weight, idx):
    n = idx.shape[0]
    d = weight.shape[1]
    b_per_w = n // _NW
    mesh = plsc.VectorSubcoreMesh(core_axis_name="c", subcore_axis_name="s")

    half = b_per_w // 2

    @functools.partial(
        pl.kernel, mesh=mesh,
        compiler_params=pltpu.CompilerParams(
            needs_layout_passes=False, use_tc_tiling_on_sc=False),
        out_type=jax.ShapeDtypeStruct((n, d), jnp.float32),
        scratch_types=[
            pltpu.VMEM((b_per_w,), jnp.int32),
            pltpu.VMEM((half, d), jnp.float32),
            pltpu.VMEM((half, d), jnp.float32),
            pltpu.SemaphoreType.DMA,
            pltpu.SemaphoreType.DMA,
            pltpu.SemaphoreType.DMA,
            pltpu.SemaphoreType.DMA,
        ],
    )
    def k(table_hbm, idx_hbm, out_hbm, idx_v, rows_a, rows_b,
          sem_a, sem_b, sem_wa, sem_wb):
        wid = lax.axis_index("s") * _NC + lax.axis_index("c")
        base = wid * b_per_w
        pltpu.sync_copy(idx_hbm.at[pl.ds(base, b_per_w)], idx_v)
        ga = pltpu.async_copy(
            table_hbm.at[idx_v.at[pl.ds(0, half)]], rows_a, sem_a)
        gb = pltpu.async_copy(
            table_hbm.at[idx_v.at[pl.ds(half, half)]], rows_b, sem_b)
        ga.wait()
        wa = pltpu.async_copy(
            rows_a, out_hbm.at[pl.ds(base, half)], sem_wa)
        gb.wait()
        wb = pltpu.async_copy(
            rows_b, out_hbm.at[pl.ds(base + half, half)], sem_wb)
        wa.wait()
        wb.wait()

    return k(weight, idx)


@jax.jit
def _vq(inputs, weight):
    d = weight.shape[1]
    xt = inputs.transpose(0, 2, 1)
    idx, loss_sum = _tc_argmin(xt, weight)
    q = _sc_gather(weight, idx.reshape(-1))
    mean_sq = loss_sum[0, 0] / jnp.float32(inputs.size)
    return (q.reshape(inputs.shape), mean_sq, _COMMITMENT_COST * mean_sq)


def kernel(inputs, weight):
    return _vq(inputs, weight)
